# Initial kernel scaffold; baseline (speedup 1.0000x reference)
import jax
import jax.numpy as jnp
from jax.experimental import pallas as pl


def kernel(x, edge_index, edge_attr, Wq, bq, Wk, Wv, We, be, WOh, bOh, WOe, bOe, g1h, b1h, g1e, b1e, Wff1, bff1, Wff2, bff2, g2h, b2h):
    raise NotImplementedError("write your pallas kernel here")


# speedup vs baseline: 12.0937x; 12.0937x over previous
import functools

import jax
import jax.numpy as jnp
import numpy as np
from jax.experimental import pallas as pl
from jax.experimental.pallas import tpu as pltpu

N = 10000
E = 160000
D = 256
H = 8
DH = 32
CLAMP = 5.0
BN_EPS = 1e-5

BN_NODE = 1000
BE = 2000


def _proj_body(x_ref, wq_ref, bq_ref, wkv_ref, q_ref, kv_ref):
    x = x_ref[...]
    q_ref[...] = jnp.dot(x, wq_ref[...],
                         preferred_element_type=jnp.float32) + bq_ref[...]
    kv_ref[...] = jnp.dot(x, wkv_ref[...], preferred_element_type=jnp.float32)


def _edge_body(ea_ref, kg_ref, qg_ref, wep_ref, bep_ref, woe_ref, boe_ref,
               msum_ref, score_ref, epre_ref, p_ref, stats_ref):
    i = pl.program_id(0)
    ea = ea_ref[...]
    eh = jnp.dot(ea, wep_ref[...], preferred_element_type=jnp.float32) \
        + bep_ref[...]
    score = kg_ref[...] * qg_ref[...] * eh[:, :D] + eh[:, D:]
    score_ref[...] = score
    s8 = jnp.dot(score, msum_ref[...], preferred_element_type=jnp.float32)
    p = jnp.exp(jnp.clip(s8, -CLAMP, CLAMP))
    p_ref[...] = jnp.pad(p, ((0, 0), (0, 8)))
    epre = ea + jnp.dot(score, woe_ref[...],
                        preferred_element_type=jnp.float32) + boe_ref[...]
    epre_ref[...] = epre

    @pl.when(i == 0)
    def _():
        stats_ref[...] = jnp.zeros_like(stats_ref)

    stats_ref[...] += jnp.stack([epre.sum(axis=0), (epre * epre).sum(axis=0)])


def _attn_body(p_ref, ssg_ref, vg_ref, score_ref, mexp_ref, contrib_ref):
    attn = p_ref[...][:, :H] / (ssg_ref[...][:, :H] + 1e-16)
    attn_x = jnp.dot(attn, mexp_ref[...], preferred_element_type=jnp.float32)
    contrib_ref[...] = (vg_ref[...] + score_ref[...]) * attn_x


def _hpre_body(x_ref, acc0_ref, acc1_ref, woh_ref, boh_ref, hpre_ref,
               stats_ref):
    i = pl.program_id(0)
    acc = acc0_ref[...] + acc1_ref[...]
    hpre = x_ref[...] + jnp.dot(acc, woh_ref[...],
                                preferred_element_type=jnp.float32) \
        + boh_ref[...]
    hpre_ref[...] = hpre

    @pl.when(i == 0)
    def _():
        stats_ref[...] = jnp.zeros_like(stats_ref)

    stats_ref[...] += jnp.stack([hpre.sum(axis=0), (hpre * hpre).sum(axis=0)])


def _bn(v, stats, g, b, cnt):
    mu = stats[0:1, :] / cnt
    var = stats[1:2, :] / cnt - mu * mu
    return g * (v - mu) * jax.lax.rsqrt(var + BN_EPS) + b


def _ffn_body(hpre_ref, st1_ref, g1_ref, b1_ref, w1_ref, bf1_ref, w2_ref,
              bf2_ref, h2pre_ref, stats_ref):
    i = pl.program_id(0)
    h1 = _bn(hpre_ref[...], st1_ref[...], g1_ref[...], b1_ref[...], float(N))
    hf = jnp.maximum(
        jnp.dot(h1, w1_ref[...], preferred_element_type=jnp.float32)
        + bf1_ref[...], 0.0)
    h2pre = h1 + jnp.dot(hf, w2_ref[...],
                         preferred_element_type=jnp.float32) + bf2_ref[...]
    h2pre_ref[...] = h2pre

    @pl.when(i == 0)
    def _():
        stats_ref[...] = jnp.zeros_like(stats_ref)

    stats_ref[...] += jnp.stack([h2pre.sum(axis=0),
                                 (h2pre * h2pre).sum(axis=0)])


def _bn_body_n(v_ref, st_ref, g_ref, b_ref, out_ref):
    out_ref[...] = _bn(v_ref[...], st_ref[...], g_ref[...], b_ref[...],
                       float(N))


def _bn_body_e(v_ref, st_ref, g_ref, b_ref, out_ref):
    out_ref[...] = _bn(v_ref[...], st_ref[...], g_ref[...], b_ref[...],
                       float(E))


def _row_spec(bn, cols):
    return pl.BlockSpec((bn, cols), lambda i: (i, 0))


def _rep_spec(shape):
    return pl.BlockSpec(shape, lambda i: tuple(0 for _ in shape))


_f32 = jnp.float32


def kernel(x, edge_index, edge_attr, Wq, bq, Wk, Wv, We, be, WOh, bOh, WOe,
           bOe, g1h, b1h, g1e, b1e, Wff1, bff1, Wff2, bff2, g2h, b2h):
    src = edge_index[0]
    dst = edge_index[1]

    Wkv = jnp.concatenate([Wk, Wv], axis=1)
    perm = np.concatenate([
        (np.arange(H)[:, None] * (2 * DH) + np.arange(DH)[None, :]).ravel(),
        (np.arange(H)[:, None] * (2 * DH) + DH
         + np.arange(DH)[None, :]).ravel(),
    ])
    Wep = We[:, perm]
    bep = be[perm][None, :]
    bq2 = bq[None, :]
    boe = bOe[None, :]
    boh = bOh[None, :]
    msum = jnp.asarray(
        np.repeat(np.eye(H, dtype=np.float32), DH, axis=0) / np.sqrt(DH))
    mexp = jnp.asarray(np.repeat(np.eye(H, dtype=np.float32), DH, axis=1))

    q_n, kv_n = pl.pallas_call(
        _proj_body,
        grid=(N // BN_NODE,),
        in_specs=[_row_spec(BN_NODE, D), _rep_spec((D, D)),
                  _rep_spec((1, D)), _rep_spec((D, 2 * D))],
        out_specs=[_row_spec(BN_NODE, D), _row_spec(BN_NODE, 2 * D)],
        out_shape=[jax.ShapeDtypeStruct((N, D), _f32),
                   jax.ShapeDtypeStruct((N, 2 * D), _f32)],
    )(x, Wq, bq2, Wkv)

    kvg = jnp.take(kv_n, src, axis=0)
    qg = jnp.take(q_n, dst, axis=0)
    kg = kvg[:, :D]
    vg = kvg[:, D:]

    score, epre, p16, est = pl.pallas_call(
        _edge_body,
        grid=(E // BE,),
        in_specs=[_row_spec(BE, D), _row_spec(BE, D), _row_spec(BE, D),
                  _rep_spec((D, 2 * D)), _rep_spec((1, 2 * D)),
                  _rep_spec((D, D)), _rep_spec((1, D)), _rep_spec((D, H))],
        out_specs=[_row_spec(BE, D), _row_spec(BE, D),
                   _row_spec(BE, 16), _rep_spec((2, D))],
        out_shape=[jax.ShapeDtypeStruct((E, D), _f32),
                   jax.ShapeDtypeStruct((E, D), _f32),
                   jax.ShapeDtypeStruct((E, 16), _f32),
                   jax.ShapeDtypeStruct((2, D), _f32)],
    )(edge_attr, kg, qg, Wep, bep, WOe, boe, msum)

    ssum = jax.ops.segment_sum(p16, dst, num_segments=N)
    ssg = jnp.take(ssum, dst, axis=0)

    contrib = pl.pallas_call(
        _attn_body,
        grid=(E // BE,),
        in_specs=[_row_spec(BE, 16), _row_spec(BE, 16), _row_spec(BE, D),
                  _row_spec(BE, D), _rep_spec((H, D))],
        out_specs=_row_spec(BE, D),
        out_shape=jax.ShapeDtypeStruct((E, D), _f32),
    )(p16, ssg, vg, score, mexp)

    acc = jax.ops.segment_sum(contrib, dst, num_segments=N)
    acc0 = acc
    acc1 = jnp.zeros_like(acc)

    hpre, st1 = pl.pallas_call(
        _hpre_body,
        grid=(N // BN_NODE,),
        in_specs=[_row_spec(BN_NODE, D), _row_spec(BN_NODE, D),
                  _row_spec(BN_NODE, D), _rep_spec((D, D)),
                  _rep_spec((1, D))],
        out_specs=[_row_spec(BN_NODE, D), _rep_spec((2, D))],
        out_shape=[jax.ShapeDtypeStruct((N, D), _f32),
                   jax.ShapeDtypeStruct((2, D), _f32)],
    )(x, acc0, acc1, WOh, boh)

    h2pre, st2 = pl.pallas_call(
        _ffn_body,
        grid=(N // BN_NODE,),
        in_specs=[_row_spec(BN_NODE, D), _rep_spec((2, D)),
                  _rep_spec((1, D)), _rep_spec((1, D)),
                  _rep_spec((D, 2 * D)), _rep_spec((1, 2 * D)),
                  _rep_spec((2 * D, D)), _rep_spec((1, D))],
        out_specs=[_row_spec(BN_NODE, D), _rep_spec((2, D))],
        out_shape=[jax.ShapeDtypeStruct((N, D), _f32),
                   jax.ShapeDtypeStruct((2, D), _f32)],
    )(hpre, st1, g1h[None, :], b1h[None, :], Wff1, bff1[None, :], Wff2,
      bff2[None, :])

    h = pl.pallas_call(
        _bn_body_n,
        grid=(N // BN_NODE,),
        in_specs=[_row_spec(BN_NODE, D), _rep_spec((2, D)),
                  _rep_spec((1, D)), _rep_spec((1, D))],
        out_specs=_row_spec(BN_NODE, D),
        out_shape=jax.ShapeDtypeStruct((N, D), _f32),
    )(h2pre, st2, g2h[None, :], b2h[None, :])

    e = pl.pallas_call(
        _bn_body_e,
        grid=(E // BE,),
        in_specs=[_row_spec(BE, D), _rep_spec((2, D)),
                  _rep_spec((1, D)), _rep_spec((1, D))],
        out_specs=_row_spec(BE, D),
        out_shape=jax.ShapeDtypeStruct((E, D), _f32),
    )(epre, est, g1e[None, :], b1e[None, :])

    return (h, e)


# speedup vs baseline: 26.6484x; 2.2035x over previous
import functools

import jax
import jax.numpy as jnp
import numpy as np
from jax import lax
from jax.experimental import pallas as pl
from jax.experimental.pallas import tpu as pltpu
from jax.experimental.pallas import tpu_sc as plsc

N = 10000
E = 160000
D = 256
H = 8
DH = 32
CLAMP = 5.0
BN_EPS = 1e-5

BN_NODE = 1000
BE = 2000

NPAD = 10240
STRIPE = 640
NW = 32
CHUNK = 128
NCH = 40
EPW = CHUNK * NCH
EP = NW * EPW
SINK = 10200


def _proj_body(x_ref, wq_ref, bq_ref, wkv_ref, q_ref, kv_ref):
    x = x_ref[...]
    q_ref[...] = jnp.dot(x, wq_ref[...],
                         preferred_element_type=jnp.float32) + bq_ref[...]
    kv_ref[...] = jnp.dot(x, wkv_ref[...], preferred_element_type=jnp.float32)


def _edge_body(ea_ref, kg_ref, qg_ref, wep_ref, bep_ref, woe_ref, boe_ref,
               msum_ref, score_ref, epre_ref, p_ref, stats_ref):
    i = pl.program_id(0)
    ea = ea_ref[...]
    eh = jnp.dot(ea, wep_ref[...], preferred_element_type=jnp.float32) \
        + bep_ref[...]
    score = kg_ref[...] * qg_ref[...] * eh[:, :D] + eh[:, D:]
    score_ref[...] = score
    s8 = jnp.dot(score, msum_ref[...], preferred_element_type=jnp.float32)
    p = jnp.exp(jnp.clip(s8, -CLAMP, CLAMP))
    p_ref[...] = jnp.pad(p, ((0, 0), (0, 8)))
    epre = ea + jnp.dot(score, woe_ref[...],
                        preferred_element_type=jnp.float32) + boe_ref[...]
    epre_ref[...] = epre

    @pl.when(i == 0)
    def _():
        stats_ref[...] = jnp.zeros_like(stats_ref)

    stats_ref[...] += jnp.stack([epre.sum(axis=0), (epre * epre).sum(axis=0)])


def _attn_body(p_ref, vg_ref, score_ref, mexp_ref, ca_ref, cb_ref, cc_ref):
    p16 = p_ref[...]
    p_x = jnp.dot(p16[:, :H], mexp_ref[...],
                  preferred_element_type=jnp.float32)
    contrib = (vg_ref[...] + score_ref[...]) * p_x
    ca_ref[...] = contrib[:, :128]
    cb_ref[...] = contrib[:, 128:]
    cc_ref[...] = jnp.pad(p16, ((0, 0), (0, 112)))


def _hpre_body(x_ref, acca_ref, accb_ref, sp_ref, mexp_ref, woh_ref, boh_ref,
               hpre_ref, stats_ref):
    i = pl.program_id(0)
    ssum8 = (sp_ref[0] + sp_ref[1])[:, :H]
    dexp = jnp.dot(1.0 / (ssum8 + 1e-16), mexp_ref[...],
                   preferred_element_type=jnp.float32)
    aa = (acca_ref[0] + acca_ref[1]) * dexp[:, :128]
    ab = (accb_ref[0] + accb_ref[1]) * dexp[:, 128:]
    w = woh_ref[...]
    hpre = x_ref[...] \
        + jnp.dot(aa, w[:128, :], preferred_element_type=jnp.float32) \
        + jnp.dot(ab, w[128:, :], preferred_element_type=jnp.float32) \
        + boh_ref[...]
    hpre_ref[...] = hpre

    @pl.when(i == 0)
    def _():
        stats_ref[...] = jnp.zeros_like(stats_ref)

    stats_ref[...] += jnp.stack([hpre.sum(axis=0), (hpre * hpre).sum(axis=0)])


def _bn(v, stats, g, b, cnt):
    mu = stats[0:1, :] / cnt
    var = stats[1:2, :] / cnt - mu * mu
    return g * (v - mu) * jax.lax.rsqrt(var + BN_EPS) + b


def _ffn_body(hpre_ref, st1_ref, g1_ref, b1_ref, w1_ref, bf1_ref, w2_ref,
              bf2_ref, h2pre_ref, stats_ref):
    i = pl.program_id(0)
    h1 = _bn(hpre_ref[...], st1_ref[...], g1_ref[...], b1_ref[...], float(N))
    hf = jnp.maximum(
        jnp.dot(h1, w1_ref[...], preferred_element_type=jnp.float32)
        + bf1_ref[...], 0.0)
    h2pre = h1 + jnp.dot(hf, w2_ref[...],
                         preferred_element_type=jnp.float32) + bf2_ref[...]
    h2pre_ref[...] = h2pre

    @pl.when(i == 0)
    def _():
        stats_ref[...] = jnp.zeros_like(stats_ref)

    stats_ref[...] += jnp.stack([h2pre.sum(axis=0),
                                 (h2pre * h2pre).sum(axis=0)])


def _bn_body_n(v_ref, st_ref, g_ref, b_ref, out_ref):
    out_ref[...] = _bn(v_ref[...], st_ref[...], g_ref[...], b_ref[...],
                       float(N))


def _bn_body_e(v_ref, st_ref, g_ref, b_ref, out_ref):
    out_ref[...] = _bn(v_ref[...], st_ref[...], g_ref[...], b_ref[...],
                       float(E))


def _row_spec(bn, cols):
    return pl.BlockSpec((bn, cols), lambda i: (i, 0))


def _rep_spec(shape):
    return pl.BlockSpec(shape, lambda i: tuple(0 for _ in shape))


_f32 = jnp.float32
_MESH = plsc.VectorSubcoreMesh(core_axis_name="c", subcore_axis_name="s")


def _sc_gather_kvq(kv, q, src3, dst3):
    @functools.partial(
        pl.kernel, mesh=_MESH,
        out_type=[jax.ShapeDtypeStruct((EP, 2 * D), _f32),
                  jax.ShapeDtypeStruct((EP, D), _f32)],
        scratch_types=[pltpu.VMEM((NCH, CHUNK), jnp.int32),
                       pltpu.VMEM((NCH, CHUNK), jnp.int32),
                       pltpu.VMEM((CHUNK, 2 * D), _f32),
                       pltpu.VMEM((CHUNK, D), _f32),
                       pltpu.SemaphoreType.DMA,
                       pltpu.SemaphoreType.DMA],
    )
    def k(kv_hbm, q_hbm, src_hbm, dst_hbm, kvg_hbm, qg_hbm,
          idxs_v, idxd_v, kvbuf, qbuf, sem1, sem2):
        wid = lax.axis_index("s") * 2 + lax.axis_index("c")
        pltpu.sync_copy(src_hbm.at[wid], idxs_v)
        pltpu.sync_copy(dst_hbm.at[wid], idxd_v)

        def body(j, carry):
            base = wid * EPW + j * CHUNK
            cp1 = pltpu.async_copy(kv_hbm.at[idxs_v.at[j]], kvbuf, sem1)
            cp2 = pltpu.async_copy(q_hbm.at[idxd_v.at[j]], qbuf, sem2)
            cp1.wait()
            pltpu.sync_copy(kvbuf, kvg_hbm.at[pl.ds(base, CHUNK)])
            cp2.wait()
            pltpu.sync_copy(qbuf, qg_hbm.at[pl.ds(base, CHUNK)])
            return carry

        lax.fori_loop(0, NCH, body, 0)

    return k(kv, q, src3, dst3)


def _sc_scatter_all(ca, cb, cc, dst3, z128):
    @functools.partial(
        pl.kernel, mesh=_MESH,
        out_type=[jax.ShapeDtypeStruct((2, N, 128), _f32),
                  jax.ShapeDtypeStruct((2, N, 128), _f32),
                  jax.ShapeDtypeStruct((2, N, 128), _f32)],
        scratch_types=[pltpu.VMEM((NCH, CHUNK), jnp.int32),
                       pltpu.VMEM((CHUNK, 128), _f32),
                       pltpu.VMEM_SHARED((NPAD, 128), _f32)],
    )
    def k(ca_hbm, cb_hbm, cc_hbm, idx_hbm, z128_hbm,
          outa_hbm, outb_hbm, outc_hbm, idx_v, vals_v, table):
        cid = lax.axis_index("c")
        sid = lax.axis_index("s")
        wid = sid * 2 + cid
        pltpu.sync_copy(idx_hbm.at[wid], idx_v)
        for src_hbm, out_hbm in ((ca_hbm, outa_hbm), (cb_hbm, outb_hbm),
                                 (cc_hbm, outc_hbm)):
            pltpu.sync_copy(z128_hbm, table.at[pl.ds(sid * STRIPE, STRIPE)])
            plsc.subcore_barrier()

            def body(j, carry, src_hbm=src_hbm):
                base = wid * EPW + j * CHUNK
                pltpu.sync_copy(src_hbm.at[pl.ds(base, CHUNK)], vals_v)
                pltpu.sync_copy(vals_v, table.at[idx_v.at[j]], add=True)
                return carry

            lax.fori_loop(0, NCH, body, 0)
            plsc.subcore_barrier()

            @pl.when(sid < 15)
            def _():
                pltpu.sync_copy(table.at[pl.ds(sid * STRIPE, STRIPE)],
                                out_hbm.at[cid, pl.ds(sid * STRIPE, STRIPE)])

            @pl.when(sid == 15)
            def _():
                pltpu.sync_copy(table.at[pl.ds(15 * STRIPE, N - 15 * STRIPE)],
                                out_hbm.at[cid, pl.ds(15 * STRIPE,
                                                      N - 15 * STRIPE)])

            plsc.subcore_barrier()

    return k(ca, cb, cc, dst3, z128)


def kernel(x, edge_index, edge_attr, Wq, bq, Wk, Wv, We, be, WOh, bOh, WOe,
           bOe, g1h, b1h, g1e, b1e, Wff1, bff1, Wff2, bff2, g2h, b2h):
    src = edge_index[0]
    dst = edge_index[1]
    src3 = jnp.pad(src, (0, EP - E)).reshape(NW, NCH, CHUNK)
    dst3 = jnp.pad(dst, (0, EP - E)).reshape(NW, NCH, CHUNK)
    dst3s = jnp.pad(dst, (0, EP - E),
                    constant_values=SINK).reshape(NW, NCH, CHUNK)
    z128 = jnp.zeros((STRIPE, 128), _f32)

    Wkv = jnp.concatenate([Wk, Wv], axis=1)
    perm = np.concatenate([
        (np.arange(H)[:, None] * (2 * DH) + np.arange(DH)[None, :]).ravel(),
        (np.arange(H)[:, None] * (2 * DH) + DH
         + np.arange(DH)[None, :]).ravel(),
    ])
    Wep = We[:, perm]
    bep = be[perm][None, :]
    bq2 = bq[None, :]
    boe = bOe[None, :]
    boh = bOh[None, :]
    msum = jnp.asarray(
        np.repeat(np.eye(H, dtype=np.float32), DH, axis=0) / np.sqrt(DH))
    mexp = jnp.asarray(np.repeat(np.eye(H, dtype=np.float32), DH, axis=1))

    q_n, kv_n = pl.pallas_call(
        _proj_body,
        grid=(N // BN_NODE,),
        in_specs=[_row_spec(BN_NODE, D), _rep_spec((D, D)),
                  _rep_spec((1, D)), _rep_spec((D, 2 * D))],
        out_specs=[_row_spec(BN_NODE, D), _row_spec(BN_NODE, 2 * D)],
        out_shape=[jax.ShapeDtypeStruct((N, D), _f32),
                   jax.ShapeDtypeStruct((N, 2 * D), _f32)],
    )(x, Wq, bq2, Wkv)

    kvg, qg = _sc_gather_kvq(kv_n, q_n, src3, dst3)

    score, epre, p16, est = pl.pallas_call(
        _edge_body,
        grid=(E // BE,),
        in_specs=[_row_spec(BE, D),
                  pl.BlockSpec((BE, D), lambda i: (i, 0)),
                  _row_spec(BE, D),
                  _rep_spec((D, 2 * D)), _rep_spec((1, 2 * D)),
                  _rep_spec((D, D)), _rep_spec((1, D)), _rep_spec((D, H))],
        out_specs=[_row_spec(BE, D), _row_spec(BE, D),
                   _row_spec(BE, 16), _rep_spec((2, D))],
        out_shape=[jax.ShapeDtypeStruct((E, D), _f32),
                   jax.ShapeDtypeStruct((E, D), _f32),
                   jax.ShapeDtypeStruct((EP, 16), _f32),
                   jax.ShapeDtypeStruct((2, D), _f32)],
    )(edge_attr, kvg, qg, Wep, bep, WOe, boe, msum)

    ca, cb, cc = pl.pallas_call(
        _attn_body,
        grid=(E // BE,),
        in_specs=[_row_spec(BE, 16),
                  pl.BlockSpec((BE, D), lambda i: (i, 1)),
                  _row_spec(BE, D), _rep_spec((H, D))],
        out_specs=[_row_spec(BE, 128), _row_spec(BE, 128),
                   _row_spec(BE, 128)],
        out_shape=[jax.ShapeDtypeStruct((EP, 128), _f32),
                   jax.ShapeDtypeStruct((EP, 128), _f32),
                   jax.ShapeDtypeStruct((EP, 128), _f32)],
    )(p16, kvg, score, mexp)

    acca, accb, sp = _sc_scatter_all(ca, cb, cc, dst3s, z128)

    hpre, st1 = pl.pallas_call(
        _hpre_body,
        grid=(N // BN_NODE,),
        in_specs=[_row_spec(BN_NODE, D),
                  pl.BlockSpec((2, BN_NODE, 128), lambda i: (0, i, 0)),
                  pl.BlockSpec((2, BN_NODE, 128), lambda i: (0, i, 0)),
                  pl.BlockSpec((2, BN_NODE, 128), lambda i: (0, i, 0)),
                  _rep_spec((H, D)), _rep_spec((D, D)), _rep_spec((1, D))],
        out_specs=[_row_spec(BN_NODE, D), _rep_spec((2, D))],
        out_shape=[jax.ShapeDtypeStruct((N, D), _f32),
                   jax.ShapeDtypeStruct((2, D), _f32)],
    )(x, acca, accb, sp, mexp, WOh, boh)

    h2pre, st2 = pl.pallas_call(
        _ffn_body,
        grid=(N // BN_NODE,),
        in_specs=[_row_spec(BN_NODE, D), _rep_spec((2, D)),
                  _rep_spec((1, D)), _rep_spec((1, D)),
                  _rep_spec((D, 2 * D)), _rep_spec((1, 2 * D)),
                  _rep_spec((2 * D, D)), _rep_spec((1, D))],
        out_specs=[_row_spec(BN_NODE, D), _rep_spec((2, D))],
        out_shape=[jax.ShapeDtypeStruct((N, D), _f32),
                   jax.ShapeDtypeStruct((2, D), _f32)],
    )(hpre, st1, g1h[None, :], b1h[None, :], Wff1, bff1[None, :], Wff2,
      bff2[None, :])

    h = pl.pallas_call(
        _bn_body_n,
        grid=(N // BN_NODE,),
        in_specs=[_row_spec(BN_NODE, D), _rep_spec((2, D)),
                  _rep_spec((1, D)), _rep_spec((1, D))],
        out_specs=_row_spec(BN_NODE, D),
        out_shape=jax.ShapeDtypeStruct((N, D), _f32),
    )(h2pre, st2, g2h[None, :], b2h[None, :])

    e = pl.pallas_call(
        _bn_body_e,
        grid=(E // BE,),
        in_specs=[_row_spec(BE, D), _rep_spec((2, D)),
                  _rep_spec((1, D)), _rep_spec((1, D))],
        out_specs=_row_spec(BE, D),
        out_shape=jax.ShapeDtypeStruct((E, D), _f32),
    )(epre, est, g1e[None, :], b1e[None, :])

    return (h, e)


# speedup vs baseline: 28.8023x; 1.0808x over previous
import functools

import jax
import jax.numpy as jnp
import numpy as np
from jax import lax
from jax.experimental import pallas as pl
from jax.experimental.pallas import tpu as pltpu
from jax.experimental.pallas import tpu_sc as plsc

N = 10000
E = 160000
D = 256
H = 8
DH = 32
CLAMP = 5.0
BN_EPS = 1e-5

BN_NODE = 1000
BE = 2000

NPAD = 10240
STRIPE = 640
NW = 32
CHUNK = 128
NCH = 40
EPW = CHUNK * NCH
EP = NW * EPW
SINK = 10200


def _proj_body(x_ref, wq_ref, bq_ref, wkv_ref, q_ref, kv_ref):
    x = x_ref[...]
    q_ref[...] = jnp.dot(x, wq_ref[...],
                         preferred_element_type=jnp.float32) + bq_ref[...]
    kv_ref[...] = jnp.dot(x, wkv_ref[...], preferred_element_type=jnp.float32)


def _edge_body(ea_ref, kvg_ref, qg_ref, wep_ref, bep_ref, woe_ref, boe_ref,
               msum_ref, mexp_ref, epre_ref, ca_ref, cb_ref, cc_ref,
               stats_ref):
    i = pl.program_id(0)
    ea = ea_ref[...]
    kvg = kvg_ref[...]
    eh = jnp.dot(ea, wep_ref[...], preferred_element_type=jnp.float32) \
        + bep_ref[...]
    score = kvg[:, :D] * qg_ref[...] * eh[:, :D] + eh[:, D:]
    s8 = jnp.dot(score, msum_ref[...], preferred_element_type=jnp.float32)
    p = jnp.exp(jnp.clip(s8, -CLAMP, CLAMP))
    p_x = jnp.dot(p, mexp_ref[...], preferred_element_type=jnp.float32)
    contrib = (kvg[:, D:] + score) * p_x
    ca_ref[...] = contrib[:, :128]
    cb_ref[...] = contrib[:, 128:]
    cc_ref[...] = jnp.pad(p, ((0, 0), (0, 120)))
    epre = ea + jnp.dot(score, woe_ref[...],
                        preferred_element_type=jnp.float32) + boe_ref[...]
    epre_ref[...] = epre

    @pl.when(i == 0)
    def _():
        stats_ref[...] = jnp.zeros_like(stats_ref)

    stats_ref[...] += jnp.stack([epre.sum(axis=0), (epre * epre).sum(axis=0)])


def _hpre_body(x_ref, acca_ref, accb_ref, sp_ref, mexp_ref, woh_ref, boh_ref,
               hpre_ref, stats_ref):
    i = pl.program_id(0)
    ssum8 = (sp_ref[0] + sp_ref[1])[:, :H]
    dexp = jnp.dot(1.0 / (ssum8 + 1e-16), mexp_ref[...],
                   preferred_element_type=jnp.float32)
    aa = (acca_ref[0] + acca_ref[1]) * dexp[:, :128]
    ab = (accb_ref[0] + accb_ref[1]) * dexp[:, 128:]
    w = woh_ref[...]
    hpre = x_ref[...] \
        + jnp.dot(aa, w[:128, :], preferred_element_type=jnp.float32) \
        + jnp.dot(ab, w[128:, :], preferred_element_type=jnp.float32) \
        + boh_ref[...]
    hpre_ref[...] = hpre

    @pl.when(i == 0)
    def _():
        stats_ref[...] = jnp.zeros_like(stats_ref)

    stats_ref[...] += jnp.stack([hpre.sum(axis=0), (hpre * hpre).sum(axis=0)])


def _bn(v, stats, g, b, cnt):
    mu = stats[0:1, :] / cnt
    var = stats[1:2, :] / cnt - mu * mu
    return g * (v - mu) * jax.lax.rsqrt(var + BN_EPS) + b


def _ffn_body(hpre_ref, st1_ref, g1_ref, b1_ref, w1_ref, bf1_ref, w2_ref,
              bf2_ref, h2pre_ref, stats_ref):
    i = pl.program_id(0)
    h1 = _bn(hpre_ref[...], st1_ref[...], g1_ref[...], b1_ref[...], float(N))
    hf = jnp.maximum(
        jnp.dot(h1, w1_ref[...], preferred_element_type=jnp.float32)
        + bf1_ref[...], 0.0)
    h2pre = h1 + jnp.dot(hf, w2_ref[...],
                         preferred_element_type=jnp.float32) + bf2_ref[...]
    h2pre_ref[...] = h2pre

    @pl.when(i == 0)
    def _():
        stats_ref[...] = jnp.zeros_like(stats_ref)

    stats_ref[...] += jnp.stack([h2pre.sum(axis=0),
                                 (h2pre * h2pre).sum(axis=0)])


def _bn_body_n(v_ref, st_ref, g_ref, b_ref, out_ref):
    out_ref[...] = _bn(v_ref[...], st_ref[...], g_ref[...], b_ref[...],
                       float(N))


def _bn_body_e(v_ref, st_ref, g_ref, b_ref, out_ref):
    out_ref[...] = _bn(v_ref[...], st_ref[...], g_ref[...], b_ref[...],
                       float(E))


def _row_spec(bn, cols):
    return pl.BlockSpec((bn, cols), lambda i: (i, 0))


def _rep_spec(shape):
    return pl.BlockSpec(shape, lambda i: tuple(0 for _ in shape))


_f32 = jnp.float32
_MESH = plsc.VectorSubcoreMesh(core_axis_name="c", subcore_axis_name="s")


def _sc_gather_kvq(kv, q, src3, dst3):
    @functools.partial(
        pl.kernel, mesh=_MESH,
        out_type=[jax.ShapeDtypeStruct((EP, 2 * D), _f32),
                  jax.ShapeDtypeStruct((EP, D), _f32)],
        scratch_types=[pltpu.VMEM((NCH, CHUNK), jnp.int32),
                       pltpu.VMEM((NCH, CHUNK), jnp.int32),
                       pltpu.VMEM((CHUNK, 2 * D), _f32),
                       pltpu.VMEM((CHUNK, D), _f32),
                       pltpu.SemaphoreType.DMA,
                       pltpu.SemaphoreType.DMA],
    )
    def k(kv_hbm, q_hbm, src_hbm, dst_hbm, kvg_hbm, qg_hbm,
          idxs_v, idxd_v, kvbuf, qbuf, sem1, sem2):
        wid = lax.axis_index("s") * 2 + lax.axis_index("c")
        pltpu.sync_copy(src_hbm.at[wid], idxs_v)
        pltpu.sync_copy(dst_hbm.at[wid], idxd_v)

        def body(j, carry):
            base = wid * EPW + j * CHUNK
            cp1 = pltpu.async_copy(kv_hbm.at[idxs_v.at[j]], kvbuf, sem1)
            cp2 = pltpu.async_copy(q_hbm.at[idxd_v.at[j]], qbuf, sem2)
            cp1.wait()
            pltpu.sync_copy(kvbuf, kvg_hbm.at[pl.ds(base, CHUNK)])
            cp2.wait()
            pltpu.sync_copy(qbuf, qg_hbm.at[pl.ds(base, CHUNK)])
            return carry

        lax.fori_loop(0, NCH, body, 0)

    return k(kv, q, src3, dst3)


def _sc_scatter_all(ca, cb, cc, dst3, z128):
    @functools.partial(
        pl.kernel, mesh=_MESH,
        out_type=[jax.ShapeDtypeStruct((2, N, 128), _f32),
                  jax.ShapeDtypeStruct((2, N, 128), _f32),
                  jax.ShapeDtypeStruct((2, N, 128), _f32)],
        scratch_types=[pltpu.VMEM((NCH, CHUNK), jnp.int32),
                       pltpu.VMEM((CHUNK, 128), _f32),
                       pltpu.VMEM_SHARED((NPAD, 128), _f32)],
    )
    def k(ca_hbm, cb_hbm, cc_hbm, idx_hbm, z128_hbm,
          outa_hbm, outb_hbm, outc_hbm, idx_v, vals_v, table):
        cid = lax.axis_index("c")
        sid = lax.axis_index("s")
        wid = sid * 2 + cid
        pltpu.sync_copy(idx_hbm.at[wid], idx_v)
        for src_hbm, out_hbm in ((ca_hbm, outa_hbm), (cb_hbm, outb_hbm),
                                 (cc_hbm, outc_hbm)):
            pltpu.sync_copy(z128_hbm, table.at[pl.ds(sid * STRIPE, STRIPE)])
            plsc.subcore_barrier()

            def body(j, carry, src_hbm=src_hbm):
                base = wid * EPW + j * CHUNK
                pltpu.sync_copy(src_hbm.at[pl.ds(base, CHUNK)], vals_v)
                pltpu.sync_copy(vals_v, table.at[idx_v.at[j]], add=True)
                return carry

            lax.fori_loop(0, NCH, body, 0)
            plsc.subcore_barrier()

            @pl.when(sid < 15)
            def _():
                pltpu.sync_copy(table.at[pl.ds(sid * STRIPE, STRIPE)],
                                out_hbm.at[cid, pl.ds(sid * STRIPE, STRIPE)])

            @pl.when(sid == 15)
            def _():
                pltpu.sync_copy(table.at[pl.ds(15 * STRIPE, N - 15 * STRIPE)],
                                out_hbm.at[cid, pl.ds(15 * STRIPE,
                                                      N - 15 * STRIPE)])

            plsc.subcore_barrier()

    return k(ca, cb, cc, dst3, z128)


def kernel(x, edge_index, edge_attr, Wq, bq, Wk, Wv, We, be, WOh, bOh, WOe,
           bOe, g1h, b1h, g1e, b1e, Wff1, bff1, Wff2, bff2, g2h, b2h):
    src = edge_index[0]
    dst = edge_index[1]
    src3 = jnp.pad(src, (0, EP - E)).reshape(NW, NCH, CHUNK)
    dst3 = jnp.pad(dst, (0, EP - E)).reshape(NW, NCH, CHUNK)
    dst3s = jnp.pad(dst, (0, EP - E),
                    constant_values=SINK).reshape(NW, NCH, CHUNK)
    z128 = jnp.zeros((STRIPE, 128), _f32)

    Wkv = jnp.concatenate([Wk, Wv], axis=1)
    perm = np.concatenate([
        (np.arange(H)[:, None] * (2 * DH) + np.arange(DH)[None, :]).ravel(),
        (np.arange(H)[:, None] * (2 * DH) + DH
         + np.arange(DH)[None, :]).ravel(),
    ])
    Wep = We[:, perm]
    bep = be[perm][None, :]
    bq2 = bq[None, :]
    boe = bOe[None, :]
    boh = bOh[None, :]
    msum = jnp.asarray(
        np.repeat(np.eye(H, dtype=np.float32), DH, axis=0) / np.sqrt(DH))
    mexp = jnp.asarray(np.repeat(np.eye(H, dtype=np.float32), DH, axis=1))

    q_n, kv_n = pl.pallas_call(
        _proj_body,
        grid=(N // BN_NODE,),
        in_specs=[_row_spec(BN_NODE, D), _rep_spec((D, D)),
                  _rep_spec((1, D)), _rep_spec((D, 2 * D))],
        out_specs=[_row_spec(BN_NODE, D), _row_spec(BN_NODE, 2 * D)],
        out_shape=[jax.ShapeDtypeStruct((N, D), _f32),
                   jax.ShapeDtypeStruct((N, 2 * D), _f32)],
    )(x, Wq, bq2, Wkv)

    kvg, qg = _sc_gather_kvq(kv_n, q_n, src3, dst3)

    epre, ca, cb, cc, est = pl.pallas_call(
        _edge_body,
        grid=(E // BE,),
        in_specs=[_row_spec(BE, D), _row_spec(BE, 2 * D), _row_spec(BE, D),
                  _rep_spec((D, 2 * D)), _rep_spec((1, 2 * D)),
                  _rep_spec((D, D)), _rep_spec((1, D)), _rep_spec((D, H)),
                  _rep_spec((H, D))],
        out_specs=[_row_spec(BE, D), _row_spec(BE, 128), _row_spec(BE, 128),
                   _row_spec(BE, 128), _rep_spec((2, D))],
        out_shape=[jax.ShapeDtypeStruct((E, D), _f32),
                   jax.ShapeDtypeStruct((EP, 128), _f32),
                   jax.ShapeDtypeStruct((EP, 128), _f32),
                   jax.ShapeDtypeStruct((EP, 128), _f32),
                   jax.ShapeDtypeStruct((2, D), _f32)],
    )(edge_attr, kvg, qg, Wep, bep, WOe, boe, msum, mexp)

    acca, accb, sp = _sc_scatter_all(ca, cb, cc, dst3s, z128)

    hpre, st1 = pl.pallas_call(
        _hpre_body,
        grid=(N // BN_NODE,),
        in_specs=[_row_spec(BN_NODE, D),
                  pl.BlockSpec((2, BN_NODE, 128), lambda i: (0, i, 0)),
                  pl.BlockSpec((2, BN_NODE, 128), lambda i: (0, i, 0)),
                  pl.BlockSpec((2, BN_NODE, 128), lambda i: (0, i, 0)),
                  _rep_spec((H, D)), _rep_spec((D, D)), _rep_spec((1, D))],
        out_specs=[_row_spec(BN_NODE, D), _rep_spec((2, D))],
        out_shape=[jax.ShapeDtypeStruct((N, D), _f32),
                   jax.ShapeDtypeStruct((2, D), _f32)],
    )(x, acca, accb, sp, mexp, WOh, boh)

    h2pre, st2 = pl.pallas_call(
        _ffn_body,
        grid=(N // BN_NODE,),
        in_specs=[_row_spec(BN_NODE, D), _rep_spec((2, D)),
                  _rep_spec((1, D)), _rep_spec((1, D)),
                  _rep_spec((D, 2 * D)), _rep_spec((1, 2 * D)),
                  _rep_spec((2 * D, D)), _rep_spec((1, D))],
        out_specs=[_row_spec(BN_NODE, D), _rep_spec((2, D))],
        out_shape=[jax.ShapeDtypeStruct((N, D), _f32),
                   jax.ShapeDtypeStruct((2, D), _f32)],
    )(hpre, st1, g1h[None, :], b1h[None, :], Wff1, bff1[None, :], Wff2,
      bff2[None, :])

    h = pl.pallas_call(
        _bn_body_n,
        grid=(N // BN_NODE,),
        in_specs=[_row_spec(BN_NODE, D), _rep_spec((2, D)),
                  _rep_spec((1, D)), _rep_spec((1, D))],
        out_specs=_row_spec(BN_NODE, D),
        out_shape=jax.ShapeDtypeStruct((N, D), _f32),
    )(h2pre, st2, g2h[None, :], b2h[None, :])

    e = pl.pallas_call(
        _bn_body_e,
        grid=(E // BE,),
        in_specs=[_row_spec(BE, D), _rep_spec((2, D)),
                  _rep_spec((1, D)), _rep_spec((1, D))],
        out_specs=_row_spec(BE, D),
        out_shape=jax.ShapeDtypeStruct((E, D), _f32),
    )(epre, est, g1e[None, :], b1e[None, :])

    return (h, e)


# speedup vs baseline: 34.2978x; 1.1908x over previous
import functools

import jax
import jax.numpy as jnp
import numpy as np
from jax import lax
from jax.experimental import pallas as pl
from jax.experimental.pallas import tpu as pltpu
from jax.experimental.pallas import tpu_sc as plsc

N = 10000
E = 160000
D = 256
H = 8
DH = 32
CLAMP = 5.0
BN_EPS = 1e-5

BN_NODE = 1000
BE = 2000

NPAD = 10240
STRIPE = 640
NW = 32
CHUNK = 128
NCH = 40
EPW = CHUNK * NCH
EP = NW * EPW
SINK = 10200


def _rne_bf16_bits(f):
    u = jax.lax.bitcast_convert_type(f, jnp.uint32)
    return (u + 0x7FFF + ((u >> 16) & 1)) >> 16


def _proj_body(x_ref, wq_ref, bq_ref, wk_ref, wv_ref, q_ref, kv_ref):
    x = x_ref[...]
    q_ref[...] = jnp.dot(x, wq_ref[...],
                         preferred_element_type=jnp.float32) + bq_ref[...]
    kb = _rne_bf16_bits(jnp.dot(x, wk_ref[...],
                                preferred_element_type=jnp.float32))
    vb = _rne_bf16_bits(jnp.dot(x, wv_ref[...],
                                preferred_element_type=jnp.float32))
    kv_ref[...] = jax.lax.bitcast_convert_type(kb | (vb << 16), jnp.int32)


def _unpack_k(p_i32):
    u = jax.lax.bitcast_convert_type(p_i32, jnp.uint32)
    return jax.lax.bitcast_convert_type(u << 16, jnp.float32)


def _unpack_v(p_i32):
    u = jax.lax.bitcast_convert_type(p_i32, jnp.uint32)
    return jax.lax.bitcast_convert_type(u & jnp.uint32(0xFFFF0000),
                                        jnp.float32)


def _edge_body(ea_ref, kvg_ref, qg_ref, wep_ref, bep_ref, woe_ref, boe_ref,
               msum_ref, mexp_ref, epre_ref, ca_ref, cb_ref, cc_ref,
               stats_ref):
    i = pl.program_id(0)
    ea = ea_ref[...]
    kvg = kvg_ref[...]
    eh = jnp.dot(ea, wep_ref[...], preferred_element_type=jnp.float32) \
        + bep_ref[...]
    score = _unpack_k(kvg) * qg_ref[...] * eh[:, :D] + eh[:, D:]
    s8 = jnp.dot(score, msum_ref[...], preferred_element_type=jnp.float32)
    p = jnp.exp(jnp.clip(s8, -CLAMP, CLAMP))
    p_x = jnp.dot(p, mexp_ref[...], preferred_element_type=jnp.float32)
    contrib = (_unpack_v(kvg) + score) * p_x
    ca_ref[...] = contrib[:, :128]
    cb_ref[...] = contrib[:, 128:]
    cc_ref[...] = jnp.pad(p, ((0, 0), (0, 120)))
    epre = ea + jnp.dot(score, woe_ref[...],
                        preferred_element_type=jnp.float32) + boe_ref[...]
    epre_ref[...] = epre

    @pl.when(i == 0)
    def _():
        stats_ref[...] = jnp.zeros_like(stats_ref)

    stats_ref[...] += jnp.stack([epre.sum(axis=0), (epre * epre).sum(axis=0)])


def _hpre_body(x_ref, acca_ref, accb_ref, sp_ref, mexp_ref, woh_ref, boh_ref,
               hpre_ref, stats_ref):
    i = pl.program_id(0)
    ssum8 = (sp_ref[0] + sp_ref[1])[:, :H]
    dexp = jnp.dot(1.0 / (ssum8 + 1e-16), mexp_ref[...],
                   preferred_element_type=jnp.float32)
    aa = (acca_ref[0] + acca_ref[1]) * dexp[:, :128]
    ab = (accb_ref[0] + accb_ref[1]) * dexp[:, 128:]
    w = woh_ref[...]
    hpre = x_ref[...] \
        + jnp.dot(aa, w[:128, :], preferred_element_type=jnp.float32) \
        + jnp.dot(ab, w[128:, :], preferred_element_type=jnp.float32) \
        + boh_ref[...]
    hpre_ref[...] = hpre

    @pl.when(i == 0)
    def _():
        stats_ref[...] = jnp.zeros_like(stats_ref)

    stats_ref[...] += jnp.stack([hpre.sum(axis=0), (hpre * hpre).sum(axis=0)])


def _bn(v, stats, g, b, cnt):
    mu = stats[0:1, :] / cnt
    var = stats[1:2, :] / cnt - mu * mu
    return g * (v - mu) * jax.lax.rsqrt(var + BN_EPS) + b


def _ffn_body(hpre_ref, st1_ref, g1_ref, b1_ref, w1_ref, bf1_ref, w2_ref,
              bf2_ref, h2pre_ref, stats_ref):
    i = pl.program_id(0)
    h1 = _bn(hpre_ref[...], st1_ref[...], g1_ref[...], b1_ref[...], float(N))
    hf = jnp.maximum(
        jnp.dot(h1, w1_ref[...], preferred_element_type=jnp.float32)
        + bf1_ref[...], 0.0)
    h2pre = h1 + jnp.dot(hf, w2_ref[...],
                         preferred_element_type=jnp.float32) + bf2_ref[...]
    h2pre_ref[...] = h2pre

    @pl.when(i == 0)
    def _():
        stats_ref[...] = jnp.zeros_like(stats_ref)

    stats_ref[...] += jnp.stack([h2pre.sum(axis=0),
                                 (h2pre * h2pre).sum(axis=0)])


def _bn_body_n(v_ref, st_ref, g_ref, b_ref, out_ref):
    out_ref[...] = _bn(v_ref[...], st_ref[...], g_ref[...], b_ref[...],
                       float(N))


def _bn_body_e(v_ref, st_ref, g_ref, b_ref, out_ref):
    out_ref[...] = _bn(v_ref[...], st_ref[...], g_ref[...], b_ref[...],
                       float(E))


def _row_spec(bn, cols):
    return pl.BlockSpec((bn, cols), lambda i: (i, 0))


def _rep_spec(shape):
    return pl.BlockSpec(shape, lambda i: tuple(0 for _ in shape))


_f32 = jnp.float32
_MESH = plsc.VectorSubcoreMesh(core_axis_name="c", subcore_axis_name="s")


def _sc_gather_kvq(kv, q, src3, dst3):
    @functools.partial(
        pl.kernel, mesh=_MESH,
        out_type=[jax.ShapeDtypeStruct((EP, D), jnp.int32),
                  jax.ShapeDtypeStruct((EP, D), _f32)],
        scratch_types=[pltpu.VMEM((NCH, CHUNK), jnp.int32),
                       pltpu.VMEM((NCH, CHUNK), jnp.int32),
                       pltpu.VMEM((CHUNK, D), jnp.int32),
                       pltpu.VMEM((CHUNK, D), _f32),
                       pltpu.SemaphoreType.DMA,
                       pltpu.SemaphoreType.DMA],
    )
    def k(kv_hbm, q_hbm, src_hbm, dst_hbm, kvg_hbm, qg_hbm,
          idxs_v, idxd_v, kvbuf, qbuf, sem1, sem2):
        wid = lax.axis_index("s") * 2 + lax.axis_index("c")
        pltpu.sync_copy(src_hbm.at[wid], idxs_v)
        pltpu.sync_copy(dst_hbm.at[wid], idxd_v)

        def body(j, carry):
            base = wid * EPW + j * CHUNK
            cp1 = pltpu.async_copy(kv_hbm.at[idxs_v.at[j]], kvbuf, sem1)
            cp2 = pltpu.async_copy(q_hbm.at[idxd_v.at[j]], qbuf, sem2)
            cp1.wait()
            pltpu.sync_copy(kvbuf, kvg_hbm.at[pl.ds(base, CHUNK)])
            cp2.wait()
            pltpu.sync_copy(qbuf, qg_hbm.at[pl.ds(base, CHUNK)])
            return carry

        lax.fori_loop(0, NCH, body, 0)

    return k(kv, q, src3, dst3)


def _sc_scatter_all(ca, cb, cc, dst3, z128):
    @functools.partial(
        pl.kernel, mesh=_MESH,
        out_type=[jax.ShapeDtypeStruct((2, N, 128), _f32),
                  jax.ShapeDtypeStruct((2, N, 128), _f32),
                  jax.ShapeDtypeStruct((2, N, 128), _f32)],
        scratch_types=[pltpu.VMEM((NCH, CHUNK), jnp.int32),
                       pltpu.VMEM((CHUNK, 128), _f32),
                       pltpu.VMEM_SHARED((NPAD, 128), _f32)],
    )
    def k(ca_hbm, cb_hbm, cc_hbm, idx_hbm, z128_hbm,
          outa_hbm, outb_hbm, outc_hbm, idx_v, vals_v, table):
        cid = lax.axis_index("c")
        sid = lax.axis_index("s")
        wid = sid * 2 + cid
        pltpu.sync_copy(idx_hbm.at[wid], idx_v)
        for src_hbm, out_hbm in ((ca_hbm, outa_hbm), (cb_hbm, outb_hbm),
                                 (cc_hbm, outc_hbm)):
            pltpu.sync_copy(z128_hbm, table.at[pl.ds(sid * STRIPE, STRIPE)])
            plsc.subcore_barrier()

            def body(j, carry, src_hbm=src_hbm):
                base = wid * EPW + j * CHUNK
                pltpu.sync_copy(src_hbm.at[pl.ds(base, CHUNK)], vals_v)
                pltpu.sync_copy(vals_v, table.at[idx_v.at[j]], add=True)
                return carry

            lax.fori_loop(0, NCH, body, 0)
            plsc.subcore_barrier()

            @pl.when(sid < 15)
            def _():
                pltpu.sync_copy(table.at[pl.ds(sid * STRIPE, STRIPE)],
                                out_hbm.at[cid, pl.ds(sid * STRIPE, STRIPE)])

            @pl.when(sid == 15)
            def _():
                pltpu.sync_copy(table.at[pl.ds(15 * STRIPE, N - 15 * STRIPE)],
                                out_hbm.at[cid, pl.ds(15 * STRIPE,
                                                      N - 15 * STRIPE)])

            plsc.subcore_barrier()

    return k(ca, cb, cc, dst3, z128)


def kernel(x, edge_index, edge_attr, Wq, bq, Wk, Wv, We, be, WOh, bOh, WOe,
           bOe, g1h, b1h, g1e, b1e, Wff1, bff1, Wff2, bff2, g2h, b2h):
    src = edge_index[0]
    dst = edge_index[1]
    src3 = jnp.pad(src, (0, EP - E)).reshape(NW, NCH, CHUNK)
    dst3 = jnp.pad(dst, (0, EP - E)).reshape(NW, NCH, CHUNK)
    dst3s = jnp.pad(dst, (0, EP - E),
                    constant_values=SINK).reshape(NW, NCH, CHUNK)
    z128 = jnp.zeros((STRIPE, 128), _f32)

    perm = np.concatenate([
        (np.arange(H)[:, None] * (2 * DH) + np.arange(DH)[None, :]).ravel(),
        (np.arange(H)[:, None] * (2 * DH) + DH
         + np.arange(DH)[None, :]).ravel(),
    ])
    Wep = We[:, perm]
    bep = be[perm][None, :]
    bq2 = bq[None, :]
    boe = bOe[None, :]
    boh = bOh[None, :]
    msum = jnp.asarray(
        np.repeat(np.eye(H, dtype=np.float32), DH, axis=0) / np.sqrt(DH))
    mexp = jnp.asarray(np.repeat(np.eye(H, dtype=np.float32), DH, axis=1))

    q_n, kv_n = pl.pallas_call(
        _proj_body,
        grid=(N // BN_NODE,),
        in_specs=[_row_spec(BN_NODE, D), _rep_spec((D, D)),
                  _rep_spec((1, D)), _rep_spec((D, D)), _rep_spec((D, D))],
        out_specs=[_row_spec(BN_NODE, D), _row_spec(BN_NODE, D)],
        out_shape=[jax.ShapeDtypeStruct((N, D), _f32),
                   jax.ShapeDtypeStruct((N, D), jnp.int32)],
    )(x, Wq, bq2, Wk, Wv)

    kvg, qg = _sc_gather_kvq(kv_n, q_n, src3, dst3)

    epre, ca, cb, cc, est = pl.pallas_call(
        _edge_body,
        grid=(E // BE,),
        in_specs=[_row_spec(BE, D), _row_spec(BE, D), _row_spec(BE, D),
                  _rep_spec((D, 2 * D)), _rep_spec((1, 2 * D)),
                  _rep_spec((D, D)), _rep_spec((1, D)), _rep_spec((D, H)),
                  _rep_spec((H, D))],
        out_specs=[_row_spec(BE, D), _row_spec(BE, 128), _row_spec(BE, 128),
                   _row_spec(BE, 128), _rep_spec((2, D))],
        out_shape=[jax.ShapeDtypeStruct((E, D), _f32),
                   jax.ShapeDtypeStruct((EP, 128), _f32),
                   jax.ShapeDtypeStruct((EP, 128), _f32),
                   jax.ShapeDtypeStruct((EP, 128), _f32),
                   jax.ShapeDtypeStruct((2, D), _f32)],
    )(edge_attr, kvg, qg, Wep, bep, WOe, boe, msum, mexp)

    acca, accb, sp = _sc_scatter_all(ca, cb, cc, dst3s, z128)

    hpre, st1 = pl.pallas_call(
        _hpre_body,
        grid=(N // BN_NODE,),
        in_specs=[_row_spec(BN_NODE, D),
                  pl.BlockSpec((2, BN_NODE, 128), lambda i: (0, i, 0)),
                  pl.BlockSpec((2, BN_NODE, 128), lambda i: (0, i, 0)),
                  pl.BlockSpec((2, BN_NODE, 128), lambda i: (0, i, 0)),
                  _rep_spec((H, D)), _rep_spec((D, D)), _rep_spec((1, D))],
        out_specs=[_row_spec(BN_NODE, D), _rep_spec((2, D))],
        out_shape=[jax.ShapeDtypeStruct((N, D), _f32),
                   jax.ShapeDtypeStruct((2, D), _f32)],
    )(x, acca, accb, sp, mexp, WOh, boh)

    h2pre, st2 = pl.pallas_call(
        _ffn_body,
        grid=(N // BN_NODE,),
        in_specs=[_row_spec(BN_NODE, D), _rep_spec((2, D)),
                  _rep_spec((1, D)), _rep_spec((1, D)),
                  _rep_spec((D, 2 * D)), _rep_spec((1, 2 * D)),
                  _rep_spec((2 * D, D)), _rep_spec((1, D))],
        out_specs=[_row_spec(BN_NODE, D), _rep_spec((2, D))],
        out_shape=[jax.ShapeDtypeStruct((N, D), _f32),
                   jax.ShapeDtypeStruct((2, D), _f32)],
    )(hpre, st1, g1h[None, :], b1h[None, :], Wff1, bff1[None, :], Wff2,
      bff2[None, :])

    h = pl.pallas_call(
        _bn_body_n,
        grid=(N // BN_NODE,),
        in_specs=[_row_spec(BN_NODE, D), _rep_spec((2, D)),
                  _rep_spec((1, D)), _rep_spec((1, D))],
        out_specs=_row_spec(BN_NODE, D),
        out_shape=jax.ShapeDtypeStruct((N, D), _f32),
    )(h2pre, st2, g2h[None, :], b2h[None, :])

    e = pl.pallas_call(
        _bn_body_e,
        grid=(E // BE,),
        in_specs=[_row_spec(BE, D), _rep_spec((2, D)),
                  _rep_spec((1, D)), _rep_spec((1, D))],
        out_specs=_row_spec(BE, D),
        out_shape=jax.ShapeDtypeStruct((E, D), _f32),
    )(epre, est, g1e[None, :], b1e[None, :])

    return (h, e)


# speedup vs baseline: 38.2355x; 1.1148x over previous
import functools

import jax
import jax.numpy as jnp
import numpy as np
from jax import lax
from jax.experimental import pallas as pl
from jax.experimental.pallas import tpu as pltpu
from jax.experimental.pallas import tpu_sc as plsc

N = 10000
E = 160000
D = 256
H = 8
DH = 32
CLAMP = 5.0
BN_EPS = 1e-5

BN_NODE = 1000
BE = 2000

NPAD = 10240
STRIPE = 640
NW = 32
CHUNK = 128
NCH = 40
EPW = CHUNK * NCH
EP = NW * EPW
SINK = 10200


def _rne_bf16_bits(f):
    u = jax.lax.bitcast_convert_type(f, jnp.uint32)
    return (u + 0x7FFF + ((u >> 16) & 1)) >> 16


def _proj_body(x_ref, wq_ref, bq_ref, wk_ref, wv_ref, q_ref, kv_ref):
    x = x_ref[...]
    q_ref[...] = jnp.dot(x, wq_ref[...],
                         preferred_element_type=jnp.float32) + bq_ref[...]
    kb = _rne_bf16_bits(jnp.dot(x, wk_ref[...],
                                preferred_element_type=jnp.float32))
    vb = _rne_bf16_bits(jnp.dot(x, wv_ref[...],
                                preferred_element_type=jnp.float32))
    kv_ref[...] = jax.lax.bitcast_convert_type(kb | (vb << 16), jnp.int32)


def _unpack_k(p_i32):
    u = jax.lax.bitcast_convert_type(p_i32, jnp.uint32)
    return jax.lax.bitcast_convert_type(u << 16, jnp.float32)


def _unpack_v(p_i32):
    u = jax.lax.bitcast_convert_type(p_i32, jnp.uint32)
    return jax.lax.bitcast_convert_type(u & jnp.uint32(0xFFFF0000),
                                        jnp.float32)


def _edge_body(ea_ref, kvg_ref, qg_ref, wep_ref, bep_ref, woe_ref, boe_ref,
               msum_ref, mexp_ref, epre_ref, ca_ref, cb_ref, cc_ref,
               stats_ref):
    i = pl.program_id(0)
    ea = ea_ref[...]
    kvg = kvg_ref[...]
    eh = jnp.dot(ea, wep_ref[...], preferred_element_type=jnp.float32) \
        + bep_ref[...]
    score = _unpack_k(kvg) * qg_ref[...] * eh[:, :D] + eh[:, D:]
    s8 = jnp.dot(score, msum_ref[...], preferred_element_type=jnp.float32)
    p = jnp.exp(jnp.clip(s8, -CLAMP, CLAMP))
    p_x = jnp.dot(p, mexp_ref[...], preferred_element_type=jnp.float32)
    contrib = (_unpack_v(kvg) + score) * p_x
    ca_ref[...] = contrib[:, :128]
    cb_ref[...] = contrib[:, 128:]
    cc_ref[...] = jnp.pad(p, ((0, 0), (0, 120)))
    epre = ea + jnp.dot(score, woe_ref[...],
                        preferred_element_type=jnp.float32) + boe_ref[...]
    epre_ref[...] = epre

    @pl.when(i == 0)
    def _():
        stats_ref[...] = jnp.zeros_like(stats_ref)

    stats_ref[...] += jnp.stack([epre.sum(axis=0), (epre * epre).sum(axis=0)])


def _hpre_body(x_ref, acca_ref, accb_ref, sp_ref, mexp_ref, woh_ref, boh_ref,
               hpre_ref, stats_ref):
    i = pl.program_id(0)
    ssum8 = (sp_ref[0] + sp_ref[1])[:, :H]
    dexp = jnp.dot(1.0 / (ssum8 + 1e-16), mexp_ref[...],
                   preferred_element_type=jnp.float32)
    aa = (acca_ref[0] + acca_ref[1]) * dexp[:, :128]
    ab = (accb_ref[0] + accb_ref[1]) * dexp[:, 128:]
    w = woh_ref[...]
    hpre = x_ref[...] \
        + jnp.dot(aa, w[:128, :], preferred_element_type=jnp.float32) \
        + jnp.dot(ab, w[128:, :], preferred_element_type=jnp.float32) \
        + boh_ref[...]
    hpre_ref[...] = hpre

    @pl.when(i == 0)
    def _():
        stats_ref[...] = jnp.zeros_like(stats_ref)

    stats_ref[...] += jnp.stack([hpre.sum(axis=0), (hpre * hpre).sum(axis=0)])


def _bn(v, stats, g, b, cnt):
    mu = stats[0:1, :] / cnt
    var = stats[1:2, :] / cnt - mu * mu
    return g * (v - mu) * jax.lax.rsqrt(var + BN_EPS) + b


def _ffn_body(hpre_ref, st1_ref, g1_ref, b1_ref, w1_ref, bf1_ref, w2_ref,
              bf2_ref, h2pre_ref, stats_ref):
    i = pl.program_id(0)
    h1 = _bn(hpre_ref[...], st1_ref[...], g1_ref[...], b1_ref[...], float(N))
    hf = jnp.maximum(
        jnp.dot(h1, w1_ref[...], preferred_element_type=jnp.float32)
        + bf1_ref[...], 0.0)
    h2pre = h1 + jnp.dot(hf, w2_ref[...],
                         preferred_element_type=jnp.float32) + bf2_ref[...]
    h2pre_ref[...] = h2pre

    @pl.when(i == 0)
    def _():
        stats_ref[...] = jnp.zeros_like(stats_ref)

    stats_ref[...] += jnp.stack([h2pre.sum(axis=0),
                                 (h2pre * h2pre).sum(axis=0)])


def _bn_body_n(v_ref, st_ref, g_ref, b_ref, out_ref):
    out_ref[...] = _bn(v_ref[...], st_ref[...], g_ref[...], b_ref[...],
                       float(N))


def _bn_body_e(v_ref, st_ref, g_ref, b_ref, out_ref):
    out_ref[...] = _bn(v_ref[...], st_ref[...], g_ref[...], b_ref[...],
                       float(E))


def _row_spec(bn, cols):
    return pl.BlockSpec((bn, cols), lambda i: (i, 0))


def _rep_spec(shape):
    return pl.BlockSpec(shape, lambda i: tuple(0 for _ in shape))


_f32 = jnp.float32
_MESH = plsc.VectorSubcoreMesh(core_axis_name="c", subcore_axis_name="s")


CHG = 64
NCHG = EPW // CHG


def _sc_gather_kvq(kv, q, src3, dst3):
    @functools.partial(
        pl.kernel, mesh=_MESH,
        out_type=[jax.ShapeDtypeStruct((EP, D), jnp.int32),
                  jax.ShapeDtypeStruct((EP, D), _f32)],
        scratch_types=[pltpu.VMEM((NCHG, CHG), jnp.int32),
                       pltpu.VMEM((NCHG, CHG), jnp.int32),
                       pltpu.VMEM((CHG, D), jnp.int32),
                       pltpu.VMEM((CHG, D), jnp.int32),
                       pltpu.VMEM((CHG, D), _f32),
                       pltpu.VMEM((CHG, D), _f32),
                       pltpu.SemaphoreType.DMA,
                       pltpu.SemaphoreType.DMA,
                       pltpu.SemaphoreType.DMA,
                       pltpu.SemaphoreType.DMA],
    )
    def k(kv_hbm, q_hbm, src_hbm, dst_hbm, kvg_hbm, qg_hbm,
          idxs_v, idxd_v, kvb0, kvb1, qb0, qb1, gk0, gk1, gq0, gq1):
        wid = lax.axis_index("s") * 2 + lax.axis_index("c")
        pltpu.sync_copy(src_hbm.at[wid], idxs_v)
        pltpu.sync_copy(dst_hbm.at[wid], idxd_v)
        kvb = (kvb0, kvb1)
        qb = (qb0, qb1)
        gk = (gk0, gk1)
        gq = (gq0, gq1)
        pltpu.async_copy(kv_hbm.at[idxs_v.at[0]], kvb0, gk0)
        pltpu.async_copy(q_hbm.at[idxd_v.at[0]], qb0, gq0)

        def body(it2, carry):
            for b in (0, 1):
                j = it2 * 2 + b
                nb = 1 - b

                @pl.when(j + 1 < NCHG)
                def _():
                    pltpu.async_copy(kv_hbm.at[idxs_v.at[j + 1]],
                                     kvb[nb], gk[nb])
                    pltpu.async_copy(q_hbm.at[idxd_v.at[j + 1]],
                                     qb[nb], gq[nb])

                base = wid * EPW + j * CHG
                pltpu.make_async_copy(kv_hbm.at[idxs_v.at[j]], kvb[b],
                                      gk[b]).wait()
                pltpu.sync_copy(kvb[b], kvg_hbm.at[pl.ds(base, CHG)])
                pltpu.make_async_copy(q_hbm.at[idxd_v.at[j]], qb[b],
                                      gq[b]).wait()
                pltpu.sync_copy(qb[b], qg_hbm.at[pl.ds(base, CHG)])
            return carry

        lax.fori_loop(0, NCHG // 2, body, 0)

    return k(kv, q, src3, dst3)


def _sc_scatter_all(ca, cb, cc, dst3, z128):
    @functools.partial(
        pl.kernel, mesh=_MESH,
        out_type=[jax.ShapeDtypeStruct((2, N, 128), _f32),
                  jax.ShapeDtypeStruct((2, N, 128), _f32),
                  jax.ShapeDtypeStruct((2, N, 128), _f32)],
        scratch_types=[pltpu.VMEM((NCH, CHUNK), jnp.int32),
                       pltpu.VMEM((CHUNK, 128), _f32),
                       pltpu.VMEM((CHUNK, 128), _f32),
                       pltpu.VMEM_SHARED((NPAD, 128), _f32),
                       pltpu.SemaphoreType.DMA,
                       pltpu.SemaphoreType.DMA],
    )
    def k(ca_hbm, cb_hbm, cc_hbm, idx_hbm, z128_hbm,
          outa_hbm, outb_hbm, outc_hbm, idx_v, vb0, vb1, table, ls0, ls1):
        cid = lax.axis_index("c")
        sid = lax.axis_index("s")
        wid = sid * 2 + cid
        vb = (vb0, vb1)
        ls = (ls0, ls1)
        pltpu.sync_copy(idx_hbm.at[wid], idx_v)
        for src_hbm, out_hbm in ((ca_hbm, outa_hbm), (cb_hbm, outb_hbm),
                                 (cc_hbm, outc_hbm)):
            pltpu.sync_copy(z128_hbm, table.at[pl.ds(sid * STRIPE, STRIPE)])
            plsc.subcore_barrier()
            pltpu.async_copy(src_hbm.at[pl.ds(wid * EPW, CHUNK)], vb0, ls0)

            def body(it2, carry, src_hbm=src_hbm):
                for b in (0, 1):
                    j = it2 * 2 + b
                    nb = 1 - b

                    @pl.when(j + 1 < NCH)
                    def _():
                        base2 = wid * EPW + (j + 1) * CHUNK
                        pltpu.async_copy(src_hbm.at[pl.ds(base2, CHUNK)],
                                         vb[nb], ls[nb])

                    base = wid * EPW + j * CHUNK
                    pltpu.make_async_copy(src_hbm.at[pl.ds(base, CHUNK)],
                                          vb[b], ls[b]).wait()
                    pltpu.sync_copy(vb[b], table.at[idx_v.at[j]], add=True)
                return carry

            lax.fori_loop(0, NCH // 2, body, 0)
            plsc.subcore_barrier()

            @pl.when(sid < 15)
            def _():
                pltpu.sync_copy(table.at[pl.ds(sid * STRIPE, STRIPE)],
                                out_hbm.at[cid, pl.ds(sid * STRIPE, STRIPE)])

            @pl.when(sid == 15)
            def _():
                pltpu.sync_copy(table.at[pl.ds(15 * STRIPE, N - 15 * STRIPE)],
                                out_hbm.at[cid, pl.ds(15 * STRIPE,
                                                      N - 15 * STRIPE)])

            plsc.subcore_barrier()

    return k(ca, cb, cc, dst3, z128)


def kernel(x, edge_index, edge_attr, Wq, bq, Wk, Wv, We, be, WOh, bOh, WOe,
           bOe, g1h, b1h, g1e, b1e, Wff1, bff1, Wff2, bff2, g2h, b2h):
    src = edge_index[0]
    dst = edge_index[1]
    src3 = jnp.pad(src, (0, EP - E)).reshape(NW, NCHG, CHG)
    dst3 = jnp.pad(dst, (0, EP - E)).reshape(NW, NCHG, CHG)
    dst3s = jnp.pad(dst, (0, EP - E),
                    constant_values=SINK).reshape(NW, NCH, CHUNK)
    z128 = jnp.zeros((STRIPE, 128), _f32)

    perm = np.concatenate([
        (np.arange(H)[:, None] * (2 * DH) + np.arange(DH)[None, :]).ravel(),
        (np.arange(H)[:, None] * (2 * DH) + DH
         + np.arange(DH)[None, :]).ravel(),
    ])
    Wep = We[:, perm]
    bep = be[perm][None, :]
    bq2 = bq[None, :]
    boe = bOe[None, :]
    boh = bOh[None, :]
    msum = jnp.asarray(
        np.repeat(np.eye(H, dtype=np.float32), DH, axis=0) / np.sqrt(DH))
    mexp = jnp.asarray(np.repeat(np.eye(H, dtype=np.float32), DH, axis=1))

    q_n, kv_n = pl.pallas_call(
        _proj_body,
        grid=(N // BN_NODE,),
        in_specs=[_row_spec(BN_NODE, D), _rep_spec((D, D)),
                  _rep_spec((1, D)), _rep_spec((D, D)), _rep_spec((D, D))],
        out_specs=[_row_spec(BN_NODE, D), _row_spec(BN_NODE, D)],
        out_shape=[jax.ShapeDtypeStruct((N, D), _f32),
                   jax.ShapeDtypeStruct((N, D), jnp.int32)],
    )(x, Wq, bq2, Wk, Wv)

    kvg, qg = _sc_gather_kvq(kv_n, q_n, src3, dst3)

    epre, ca, cb, cc, est = pl.pallas_call(
        _edge_body,
        grid=(E // BE,),
        in_specs=[_row_spec(BE, D), _row_spec(BE, D), _row_spec(BE, D),
                  _rep_spec((D, 2 * D)), _rep_spec((1, 2 * D)),
                  _rep_spec((D, D)), _rep_spec((1, D)), _rep_spec((D, H)),
                  _rep_spec((H, D))],
        out_specs=[_row_spec(BE, D), _row_spec(BE, 128), _row_spec(BE, 128),
                   _row_spec(BE, 128), _rep_spec((2, D))],
        out_shape=[jax.ShapeDtypeStruct((E, D), _f32),
                   jax.ShapeDtypeStruct((EP, 128), _f32),
                   jax.ShapeDtypeStruct((EP, 128), _f32),
                   jax.ShapeDtypeStruct((EP, 128), _f32),
                   jax.ShapeDtypeStruct((2, D), _f32)],
    )(edge_attr, kvg, qg, Wep, bep, WOe, boe, msum, mexp)

    acca, accb, sp = _sc_scatter_all(ca, cb, cc, dst3s, z128)

    hpre, st1 = pl.pallas_call(
        _hpre_body,
        grid=(N // BN_NODE,),
        in_specs=[_row_spec(BN_NODE, D),
                  pl.BlockSpec((2, BN_NODE, 128), lambda i: (0, i, 0)),
                  pl.BlockSpec((2, BN_NODE, 128), lambda i: (0, i, 0)),
                  pl.BlockSpec((2, BN_NODE, 128), lambda i: (0, i, 0)),
                  _rep_spec((H, D)), _rep_spec((D, D)), _rep_spec((1, D))],
        out_specs=[_row_spec(BN_NODE, D), _rep_spec((2, D))],
        out_shape=[jax.ShapeDtypeStruct((N, D), _f32),
                   jax.ShapeDtypeStruct((2, D), _f32)],
    )(x, acca, accb, sp, mexp, WOh, boh)

    h2pre, st2 = pl.pallas_call(
        _ffn_body,
        grid=(N // BN_NODE,),
        in_specs=[_row_spec(BN_NODE, D), _rep_spec((2, D)),
                  _rep_spec((1, D)), _rep_spec((1, D)),
                  _rep_spec((D, 2 * D)), _rep_spec((1, 2 * D)),
                  _rep_spec((2 * D, D)), _rep_spec((1, D))],
        out_specs=[_row_spec(BN_NODE, D), _rep_spec((2, D))],
        out_shape=[jax.ShapeDtypeStruct((N, D), _f32),
                   jax.ShapeDtypeStruct((2, D), _f32)],
    )(hpre, st1, g1h[None, :], b1h[None, :], Wff1, bff1[None, :], Wff2,
      bff2[None, :])

    h = pl.pallas_call(
        _bn_body_n,
        grid=(N // BN_NODE,),
        in_specs=[_row_spec(BN_NODE, D), _rep_spec((2, D)),
                  _rep_spec((1, D)), _rep_spec((1, D))],
        out_specs=_row_spec(BN_NODE, D),
        out_shape=jax.ShapeDtypeStruct((N, D), _f32),
    )(h2pre, st2, g2h[None, :], b2h[None, :])

    e = pl.pallas_call(
        _bn_body_e,
        grid=(E // BE,),
        in_specs=[_row_spec(BE, D), _rep_spec((2, D)),
                  _rep_spec((1, D)), _rep_spec((1, D))],
        out_specs=_row_spec(BE, D),
        out_shape=jax.ShapeDtypeStruct((E, D), _f32),
    )(epre, est, g1e[None, :], b1e[None, :])

    return (h, e)


# speedup vs baseline: 39.8786x; 1.0430x over previous
import functools

import jax
import jax.numpy as jnp
import numpy as np
from jax import lax
from jax.experimental import pallas as pl
from jax.experimental.pallas import tpu as pltpu
from jax.experimental.pallas import tpu_sc as plsc

N = 10000
E = 160000
D = 256
H = 8
DH = 32
CLAMP = 5.0
BN_EPS = 1e-5

BN_NODE = 1000
BE = 2000

NPAD = 10240
STRIPE = 640
NW = 32
CHUNK = 128
NCH = 40
EPW = CHUNK * NCH
EP = NW * EPW
SINK = 10200


def _rne_bf16_bits(f):
    u = jax.lax.bitcast_convert_type(f, jnp.uint32)
    return (u + 0x7FFF + ((u >> 16) & 1)) >> 16


def _proj_body(x_ref, wq_ref, bq_ref, wk_ref, wv_ref, q_ref, kv_ref):
    x = x_ref[...]
    q_ref[...] = jnp.dot(x, wq_ref[...],
                         preferred_element_type=jnp.float32) + bq_ref[...]
    kb = _rne_bf16_bits(jnp.dot(x, wk_ref[...],
                                preferred_element_type=jnp.float32))
    vb = _rne_bf16_bits(jnp.dot(x, wv_ref[...],
                                preferred_element_type=jnp.float32))
    kv_ref[...] = jax.lax.bitcast_convert_type(kb | (vb << 16), jnp.int32)


def _unpack_k(p_i32):
    u = jax.lax.bitcast_convert_type(p_i32, jnp.uint32)
    return jax.lax.bitcast_convert_type(u << 16, jnp.float32)


def _unpack_v(p_i32):
    u = jax.lax.bitcast_convert_type(p_i32, jnp.uint32)
    return jax.lax.bitcast_convert_type(u & jnp.uint32(0xFFFF0000),
                                        jnp.float32)


def _edge_body(ea_ref, kvg_ref, qg_ref, wep_ref, bep_ref, woe_ref, boe_ref,
               msum_ref, mexp_ref, epre_ref, ca_ref, cb_ref, cc_ref,
               stats_ref):
    i = pl.program_id(0)
    ea = ea_ref[...]
    kvg = kvg_ref[...]
    eh = jnp.dot(ea, wep_ref[...], preferred_element_type=jnp.float32) \
        + bep_ref[...]
    score = _unpack_k(kvg) * qg_ref[...] * eh[:, :D] + eh[:, D:]
    s8 = jnp.dot(score, msum_ref[...], preferred_element_type=jnp.float32)
    p = jnp.exp(jnp.clip(s8, -CLAMP, CLAMP))
    p_x = jnp.dot(p, mexp_ref[...], preferred_element_type=jnp.float32)
    contrib = (_unpack_v(kvg) + score) * p_x
    ca_ref[...] = contrib[:, :128]
    cb_ref[...] = contrib[:, 128:]
    cc_ref[...] = jnp.pad(p, ((0, 0), (0, 120)))
    epre = ea + jnp.dot(score, woe_ref[...],
                        preferred_element_type=jnp.float32) + boe_ref[...]
    epre_ref[...] = epre.astype(jnp.bfloat16)

    @pl.when(i == 0)
    def _():
        stats_ref[...] = jnp.zeros_like(stats_ref)

    stats_ref[...] += jnp.stack([epre.sum(axis=0), (epre * epre).sum(axis=0)])


def _hpre_body(x_ref, acca_ref, accb_ref, sp_ref, mexp_ref, woh_ref, boh_ref,
               hpre_ref, stats_ref):
    i = pl.program_id(0)
    ssum8 = (sp_ref[0] + sp_ref[1])[:, :H]
    dexp = jnp.dot(1.0 / (ssum8 + 1e-16), mexp_ref[...],
                   preferred_element_type=jnp.float32)
    aa = (acca_ref[0] + acca_ref[1]) * dexp[:, :128]
    ab = (accb_ref[0] + accb_ref[1]) * dexp[:, 128:]
    w = woh_ref[...]
    hpre = x_ref[...] \
        + jnp.dot(aa, w[:128, :], preferred_element_type=jnp.float32) \
        + jnp.dot(ab, w[128:, :], preferred_element_type=jnp.float32) \
        + boh_ref[...]
    hpre_ref[...] = hpre

    @pl.when(i == 0)
    def _():
        stats_ref[...] = jnp.zeros_like(stats_ref)

    stats_ref[...] += jnp.stack([hpre.sum(axis=0), (hpre * hpre).sum(axis=0)])


def _bn(v, stats, g, b, cnt):
    mu = stats[0:1, :] / cnt
    var = stats[1:2, :] / cnt - mu * mu
    return g * (v - mu) * jax.lax.rsqrt(var + BN_EPS) + b


def _ffn_body(hpre_ref, st1_ref, g1_ref, b1_ref, w1_ref, bf1_ref, w2_ref,
              bf2_ref, h2pre_ref, stats_ref):
    i = pl.program_id(0)
    h1 = _bn(hpre_ref[...], st1_ref[...], g1_ref[...], b1_ref[...], float(N))
    hf = jnp.maximum(
        jnp.dot(h1, w1_ref[...], preferred_element_type=jnp.float32)
        + bf1_ref[...], 0.0)
    h2pre = h1 + jnp.dot(hf, w2_ref[...],
                         preferred_element_type=jnp.float32) + bf2_ref[...]
    h2pre_ref[...] = h2pre

    @pl.when(i == 0)
    def _():
        stats_ref[...] = jnp.zeros_like(stats_ref)

    stats_ref[...] += jnp.stack([h2pre.sum(axis=0),
                                 (h2pre * h2pre).sum(axis=0)])


def _bn_body_n(v_ref, st_ref, g_ref, b_ref, out_ref):
    out_ref[...] = _bn(v_ref[...], st_ref[...], g_ref[...], b_ref[...],
                       float(N))


def _bn_body_e(v_ref, st_ref, g_ref, b_ref, out_ref):
    out_ref[...] = _bn(v_ref[...].astype(jnp.float32), st_ref[...],
                       g_ref[...], b_ref[...], float(E))


def _row_spec(bn, cols):
    return pl.BlockSpec((bn, cols), lambda i: (i, 0))


def _rep_spec(shape):
    return pl.BlockSpec(shape, lambda i: tuple(0 for _ in shape))


_f32 = jnp.float32
_MESH = plsc.VectorSubcoreMesh(core_axis_name="c", subcore_axis_name="s")


CHG = 64
NCHG = EPW // CHG


def _sc_gather_kvq(kv, q, src3, dst3):
    @functools.partial(
        pl.kernel, mesh=_MESH,
        out_type=[jax.ShapeDtypeStruct((EP, D), jnp.int32),
                  jax.ShapeDtypeStruct((EP, D), _f32)],
        scratch_types=[pltpu.VMEM((NCHG, CHG), jnp.int32),
                       pltpu.VMEM((NCHG, CHG), jnp.int32),
                       pltpu.VMEM((CHG, D), jnp.int32),
                       pltpu.VMEM((CHG, D), jnp.int32),
                       pltpu.VMEM((CHG, D), _f32),
                       pltpu.VMEM((CHG, D), _f32),
                       pltpu.SemaphoreType.DMA,
                       pltpu.SemaphoreType.DMA,
                       pltpu.SemaphoreType.DMA,
                       pltpu.SemaphoreType.DMA],
    )
    def k(kv_hbm, q_hbm, src_hbm, dst_hbm, kvg_hbm, qg_hbm,
          idxs_v, idxd_v, kvb0, kvb1, qb0, qb1, gk0, gk1, gq0, gq1):
        wid = lax.axis_index("s") * 2 + lax.axis_index("c")
        pltpu.sync_copy(src_hbm.at[wid], idxs_v)
        pltpu.sync_copy(dst_hbm.at[wid], idxd_v)
        kvb = (kvb0, kvb1)
        qb = (qb0, qb1)
        gk = (gk0, gk1)
        gq = (gq0, gq1)
        pltpu.async_copy(kv_hbm.at[idxs_v.at[0]], kvb0, gk0)
        pltpu.async_copy(q_hbm.at[idxd_v.at[0]], qb0, gq0)

        def body(it2, carry):
            for b in (0, 1):
                j = it2 * 2 + b
                nb = 1 - b

                @pl.when(j + 1 < NCHG)
                def _():
                    pltpu.async_copy(kv_hbm.at[idxs_v.at[j + 1]],
                                     kvb[nb], gk[nb])
                    pltpu.async_copy(q_hbm.at[idxd_v.at[j + 1]],
                                     qb[nb], gq[nb])

                base = wid * EPW + j * CHG
                pltpu.make_async_copy(kv_hbm.at[idxs_v.at[j]], kvb[b],
                                      gk[b]).wait()
                pltpu.sync_copy(kvb[b], kvg_hbm.at[pl.ds(base, CHG)])
                pltpu.make_async_copy(q_hbm.at[idxd_v.at[j]], qb[b],
                                      gq[b]).wait()
                pltpu.sync_copy(qb[b], qg_hbm.at[pl.ds(base, CHG)])
            return carry

        lax.fori_loop(0, NCHG // 2, body, 0)

    return k(kv, q, src3, dst3)


def _sc_scatter_all(ca, cb, cc, dst3, z128):
    @functools.partial(
        pl.kernel, mesh=_MESH,
        out_type=[jax.ShapeDtypeStruct((2, N, 128), _f32),
                  jax.ShapeDtypeStruct((2, N, 128), _f32),
                  jax.ShapeDtypeStruct((2, N, 128), _f32)],
        scratch_types=[pltpu.VMEM((NCH, CHUNK), jnp.int32),
                       pltpu.VMEM((CHUNK, 128), _f32),
                       pltpu.VMEM((CHUNK, 128), _f32),
                       pltpu.VMEM_SHARED((NPAD, 128), _f32),
                       pltpu.SemaphoreType.DMA,
                       pltpu.SemaphoreType.DMA],
    )
    def k(ca_hbm, cb_hbm, cc_hbm, idx_hbm, z128_hbm,
          outa_hbm, outb_hbm, outc_hbm, idx_v, vb0, vb1, table, ls0, ls1):
        cid = lax.axis_index("c")
        sid = lax.axis_index("s")
        wid = sid * 2 + cid
        vb = (vb0, vb1)
        ls = (ls0, ls1)
        pltpu.sync_copy(idx_hbm.at[wid], idx_v)
        for src_hbm, out_hbm in ((ca_hbm, outa_hbm), (cb_hbm, outb_hbm),
                                 (cc_hbm, outc_hbm)):
            pltpu.sync_copy(z128_hbm, table.at[pl.ds(sid * STRIPE, STRIPE)])
            plsc.subcore_barrier()
            pltpu.async_copy(src_hbm.at[pl.ds(wid * EPW, CHUNK)], vb0, ls0)

            def body(it2, carry, src_hbm=src_hbm):
                for b in (0, 1):
                    j = it2 * 2 + b
                    nb = 1 - b

                    @pl.when(j + 1 < NCH)
                    def _():
                        base2 = wid * EPW + (j + 1) * CHUNK
                        pltpu.async_copy(src_hbm.at[pl.ds(base2, CHUNK)],
                                         vb[nb], ls[nb])

                    base = wid * EPW + j * CHUNK
                    pltpu.make_async_copy(src_hbm.at[pl.ds(base, CHUNK)],
                                          vb[b], ls[b]).wait()
                    pltpu.sync_copy(vb[b], table.at[idx_v.at[j]], add=True)
                return carry

            lax.fori_loop(0, NCH // 2, body, 0)
            plsc.subcore_barrier()

            @pl.when(sid < 15)
            def _():
                pltpu.sync_copy(table.at[pl.ds(sid * STRIPE, STRIPE)],
                                out_hbm.at[cid, pl.ds(sid * STRIPE, STRIPE)])

            @pl.when(sid == 15)
            def _():
                pltpu.sync_copy(table.at[pl.ds(15 * STRIPE, N - 15 * STRIPE)],
                                out_hbm.at[cid, pl.ds(15 * STRIPE,
                                                      N - 15 * STRIPE)])

            plsc.subcore_barrier()

    return k(ca, cb, cc, dst3, z128)


def kernel(x, edge_index, edge_attr, Wq, bq, Wk, Wv, We, be, WOh, bOh, WOe,
           bOe, g1h, b1h, g1e, b1e, Wff1, bff1, Wff2, bff2, g2h, b2h):
    src = edge_index[0]
    dst = edge_index[1]
    src3 = jnp.pad(src, (0, EP - E)).reshape(NW, NCHG, CHG)
    dst3 = jnp.pad(dst, (0, EP - E)).reshape(NW, NCHG, CHG)
    dst3s = jnp.pad(dst, (0, EP - E),
                    constant_values=SINK).reshape(NW, NCH, CHUNK)
    z128 = jnp.zeros((STRIPE, 128), _f32)

    perm = np.concatenate([
        (np.arange(H)[:, None] * (2 * DH) + np.arange(DH)[None, :]).ravel(),
        (np.arange(H)[:, None] * (2 * DH) + DH
         + np.arange(DH)[None, :]).ravel(),
    ])
    Wep = We[:, perm]
    bep = be[perm][None, :]
    bq2 = bq[None, :]
    boe = bOe[None, :]
    boh = bOh[None, :]
    msum = jnp.asarray(
        np.repeat(np.eye(H, dtype=np.float32), DH, axis=0) / np.sqrt(DH))
    mexp = jnp.asarray(np.repeat(np.eye(H, dtype=np.float32), DH, axis=1))

    q_n, kv_n = pl.pallas_call(
        _proj_body,
        grid=(N // BN_NODE,),
        in_specs=[_row_spec(BN_NODE, D), _rep_spec((D, D)),
                  _rep_spec((1, D)), _rep_spec((D, D)), _rep_spec((D, D))],
        out_specs=[_row_spec(BN_NODE, D), _row_spec(BN_NODE, D)],
        out_shape=[jax.ShapeDtypeStruct((N, D), _f32),
                   jax.ShapeDtypeStruct((N, D), jnp.int32)],
    )(x, Wq, bq2, Wk, Wv)

    kvg, qg = _sc_gather_kvq(kv_n, q_n, src3, dst3)

    epre, ca, cb, cc, est = pl.pallas_call(
        _edge_body,
        grid=(E // BE,),
        in_specs=[_row_spec(BE, D), _row_spec(BE, D), _row_spec(BE, D),
                  _rep_spec((D, 2 * D)), _rep_spec((1, 2 * D)),
                  _rep_spec((D, D)), _rep_spec((1, D)), _rep_spec((D, H)),
                  _rep_spec((H, D))],
        out_specs=[_row_spec(BE, D), _row_spec(BE, 128), _row_spec(BE, 128),
                   _row_spec(BE, 128), _rep_spec((2, D))],
        out_shape=[jax.ShapeDtypeStruct((E, D), jnp.bfloat16),
                   jax.ShapeDtypeStruct((EP, 128), _f32),
                   jax.ShapeDtypeStruct((EP, 128), _f32),
                   jax.ShapeDtypeStruct((EP, 128), _f32),
                   jax.ShapeDtypeStruct((2, D), _f32)],
    )(edge_attr, kvg, qg, Wep, bep, WOe, boe, msum, mexp)

    acca, accb, sp = _sc_scatter_all(ca, cb, cc, dst3s, z128)

    hpre, st1 = pl.pallas_call(
        _hpre_body,
        grid=(N // BN_NODE,),
        in_specs=[_row_spec(BN_NODE, D),
                  pl.BlockSpec((2, BN_NODE, 128), lambda i: (0, i, 0)),
                  pl.BlockSpec((2, BN_NODE, 128), lambda i: (0, i, 0)),
                  pl.BlockSpec((2, BN_NODE, 128), lambda i: (0, i, 0)),
                  _rep_spec((H, D)), _rep_spec((D, D)), _rep_spec((1, D))],
        out_specs=[_row_spec(BN_NODE, D), _rep_spec((2, D))],
        out_shape=[jax.ShapeDtypeStruct((N, D), _f32),
                   jax.ShapeDtypeStruct((2, D), _f32)],
    )(x, acca, accb, sp, mexp, WOh, boh)

    h2pre, st2 = pl.pallas_call(
        _ffn_body,
        grid=(N // BN_NODE,),
        in_specs=[_row_spec(BN_NODE, D), _rep_spec((2, D)),
                  _rep_spec((1, D)), _rep_spec((1, D)),
                  _rep_spec((D, 2 * D)), _rep_spec((1, 2 * D)),
                  _rep_spec((2 * D, D)), _rep_spec((1, D))],
        out_specs=[_row_spec(BN_NODE, D), _rep_spec((2, D))],
        out_shape=[jax.ShapeDtypeStruct((N, D), _f32),
                   jax.ShapeDtypeStruct((2, D), _f32)],
    )(hpre, st1, g1h[None, :], b1h[None, :], Wff1, bff1[None, :], Wff2,
      bff2[None, :])

    h = pl.pallas_call(
        _bn_body_n,
        grid=(N // BN_NODE,),
        in_specs=[_row_spec(BN_NODE, D), _rep_spec((2, D)),
                  _rep_spec((1, D)), _rep_spec((1, D))],
        out_specs=_row_spec(BN_NODE, D),
        out_shape=jax.ShapeDtypeStruct((N, D), _f32),
    )(h2pre, st2, g2h[None, :], b2h[None, :])

    e = pl.pallas_call(
        _bn_body_e,
        grid=(E // BE,),
        in_specs=[_row_spec(BE, D), _rep_spec((2, D)),
                  _rep_spec((1, D)), _rep_spec((1, D))],
        out_specs=_row_spec(BE, D),
        out_shape=jax.ShapeDtypeStruct((E, D), _f32),
    )(epre, est, g1e[None, :], b1e[None, :])

    return (h, e)
